# Initial kernel scaffold; baseline (speedup 1.0000x reference)
import jax
import jax.numpy as jnp
from jax.experimental import pallas as pl


def kernel(x_seq, edge_attr_seq, edge_index, seq_len, W_node1, b_node1, W_node2, b_node2, W_edge1, b_edge1, W_edge2, b_edge2, gcn1_W, gcn1_b, gcn2_W, gcn2_b, efc1_W, efc1_b, efc2_W, efc2_b, Wih_n, Whh_n, bih_n, bhh_n, Wih_e, Whh_e, bih_e, bhh_e, Wc1, bc1, Wc2, bc2):
    raise NotImplementedError("write your pallas kernel here")


# speedup vs baseline: 2.7611x; 2.7611x over previous
import functools

import jax
import jax.numpy as jnp
from jax import lax
from jax.experimental import pallas as pl
from jax.experimental.pallas import tpu as pltpu
from jax.experimental.pallas import tpu_sc as plsc

F32 = jnp.float32

T = 4
TP = 4
N = 10000
E = 50000
D_NODE = 128
D_EDGE = 16
H_N = 128
H_E = 64
MLP_H = 256
N_CLS = 8

NC = 2
NS = 16
NW = NC * NS

STRIPE = 632
NPAD = NS * STRIPE
NB = 632

CHB = 128
KSUB = 13
CHUNK = KSUB * CHB
EPAD = NW * CHUNK
EB = 1000
EBP = 832
NBL = 632
EBL = 400


def _mm(a, b):
    return jax.lax.dot_general(a, b, (((1,), (0,)), ((), ())),
                               preferred_element_type=F32)


def _sigmoid(x):
    return 1.0 / (1.0 + jnp.exp(-x))


def _gate_scale(w, h_dim):
    s = jnp.concatenate([jnp.full((h_dim,), 0.5, F32),
                         jnp.full((h_dim,), 0.5, F32),
                         jnp.ones((h_dim,), F32),
                         jnp.full((h_dim,), 0.5, F32)])
    return w * s


def _mesh():
    return plsc.VectorSubcoreMesh(core_axis_name="c", subcore_axis_name="s",
                                  num_cores=NC, num_subcores=NS)


@functools.cache
def _get_sc_degree():
    @functools.partial(
        pl.kernel,
        out_type=jax.ShapeDtypeStruct((NC, NPAD, H_N), F32),
        mesh=_mesh(),
        scratch_types=[
            pltpu.VMEM((KSUB, CHB), jnp.int32),
            pltpu.VMEM((CHB, H_N), F32),
            pltpu.VMEM_SHARED((NPAD, H_N), F32),
        ],
    )
    def _sc_degree(dst_hbm, ones_hbm, zeros_hbm, out_hbm,
                   dst_v, ones_v, acc_sh):
        c = lax.axis_index("c")
        s = lax.axis_index("s")
        wid = c * NS + s
        pltpu.sync_copy(dst_hbm.at[wid], dst_v)
        pltpu.sync_copy(ones_hbm, ones_v)
        pltpu.sync_copy(zeros_hbm, acc_sh.at[pl.ds(s * STRIPE, STRIPE)])
        plsc.subcore_barrier()
        for j in range(KSUB):
            pltpu.sync_copy(ones_v, acc_sh.at[dst_v.at[j]], add=True)
        plsc.subcore_barrier()
        pltpu.sync_copy(acc_sh.at[pl.ds(s * STRIPE, STRIPE)],
                        out_hbm.at[c, pl.ds(s * STRIPE, STRIPE)])

    return _sc_degree


@functools.cache
def _get_sc_gcn():
    @functools.partial(
        pl.kernel,
        out_type=jax.ShapeDtypeStruct((NC, T, NPAD, H_N), F32),
        mesh=_mesh(),
        scratch_types=[
            pltpu.VMEM((KSUB, CHB), jnp.int32),
            pltpu.VMEM((KSUB, CHB), jnp.int32),
            pltpu.VMEM((CHB, H_N), F32),
            pltpu.VMEM_SHARED((NPAD, H_N), F32),
            pltpu.SemaphoreType.DMA,
        ],
    )
    def _sc_gcn(y0, y1, y2, y3, src_hbm, dst_hbm, zeros_hbm,
                out_hbm, src_v, dst_v, rows_v, acc_sh, sem):
        c = lax.axis_index("c")
        s = lax.axis_index("s")
        wid = c * NS + s
        pltpu.sync_copy(src_hbm.at[wid], src_v)
        pltpu.sync_copy(dst_hbm.at[wid], dst_v)
        ys = [y0, y1, y2, y3]
        for t in range(T):
            pltpu.sync_copy(zeros_hbm, acc_sh.at[pl.ds(s * STRIPE, STRIPE)])
            plsc.subcore_barrier()
            for j in range(KSUB):
                pltpu.async_copy(ys[t].at[src_v.at[j]], rows_v, sem).wait()
                pltpu.sync_copy(rows_v, acc_sh.at[dst_v.at[j]], add=True)
            plsc.subcore_barrier()
            pltpu.sync_copy(acc_sh.at[pl.ds(s * STRIPE, STRIPE)],
                            out_hbm.at[c, t, pl.ds(s * STRIPE, STRIPE)])
            plsc.subcore_barrier()

    return _sc_gcn


@functools.cache
def _get_sc_class_gather():
    @functools.partial(
        pl.kernel,
        out_type=[jax.ShapeDtypeStruct((T, EPAD, H_N), F32),
                  jax.ShapeDtypeStruct((T, EPAD, H_N), F32)],
        mesh=_mesh(),
        scratch_types=[
            pltpu.VMEM((KSUB, CHB), jnp.int32),
            pltpu.VMEM((KSUB, CHB), jnp.int32),
            pltpu.VMEM((CHB, H_N), F32),
            pltpu.VMEM((CHB, H_N), F32),
            pltpu.SemaphoreType.DMA,
            pltpu.SemaphoreType.DMA,
        ],
    )
    def _sc_class_gather(ln0, ln1, ln2, ln3, src_hbm, dst_hbm, gs_hbm,
                         gd_hbm, src_v, dst_v, rows_a, rows_b, sem_a, sem_b):
        c = lax.axis_index("c")
        s = lax.axis_index("s")
        wid = c * NS + s
        base = wid * CHUNK
        pltpu.sync_copy(src_hbm.at[wid], src_v)
        pltpu.sync_copy(dst_hbm.at[wid], dst_v)
        lns = [ln0, ln1, ln2, ln3]
        for t in range(T):
            for j in range(KSUB):
                ca = pltpu.async_copy(lns[t].at[src_v.at[j]], rows_a, sem_a)
                cb = pltpu.async_copy(lns[t].at[dst_v.at[j]], rows_b, sem_b)
                ca.wait()
                pltpu.sync_copy(rows_a,
                                gs_hbm.at[t, pl.ds(base + j * CHB, CHB)])
                cb.wait()
                pltpu.sync_copy(rows_b,
                                gd_hbm.at[t, pl.ds(base + j * CHB, CHB)])

    return _sc_class_gather


def _k_dinv(parts_ref, out_ref):
    sm = parts_ref[0] + parts_ref[1]
    deg = 1.0 + sm[:, 0:1]
    out_ref[...] = jax.lax.rsqrt(deg)


def _dinv_call(parts):
    return pl.pallas_call(
        _k_dinv,
        grid=(1,),
        in_specs=[pl.BlockSpec((NC, NPAD, H_N), lambda i: (0, 0, 0))],
        out_specs=pl.BlockSpec((NPAD, 1), lambda i: (0, 0)),
        out_shape=jax.ShapeDtypeStruct((NPAD, 1), F32),
    )(parts)


def _k_node_front(x_ref, w1_ref, b1_ref, w2_ref, b2_ref, gw_ref, dinv_ref,
                  out_ref):
    x = x_ref[0]
    h = jnp.maximum(_mm(x, w1_ref[...]) + b1_ref[...], 0.0)
    h = jnp.maximum(_mm(h, w2_ref[...]) + b2_ref[...], 0.0)
    out_ref[0] = _mm(h, gw_ref[...]) * dinv_ref[...]


def _node_front_call(xp, w1, b1, w2, b2, gw, dinv):
    full = lambda shape: pl.BlockSpec(shape, lambda t, i: tuple(0 for _ in shape))
    return pl.pallas_call(
        _k_node_front,
        grid=(T, NPAD // NB),
        in_specs=[
            pl.BlockSpec((1, NB, D_NODE), lambda t, i: (t, i, 0)),
            full((D_NODE, H_N)), full((1, H_N)),
            full((H_N, H_N)), full((1, H_N)),
            full((H_N, H_N)),
            pl.BlockSpec((NB, 1), lambda t, i: (i, 0)),
        ],
        out_specs=pl.BlockSpec((1, NB, H_N), lambda t, i: (t, i, 0)),
        out_shape=jax.ShapeDtypeStruct((T, NPAD, H_N), F32),
    )(xp, w1, b1, w2, b2, gw, dinv)


def _k_gcn_mid(parts_ref, y_ref, dinv_ref, b_ref, w2_ref, out_ref):
    ssum = parts_ref[0, 0] + parts_ref[1, 0] + y_ref[0]
    x = jnp.maximum(ssum * dinv_ref[...] + b_ref[...], 0.0)
    out_ref[0] = _mm(x, w2_ref[...]) * dinv_ref[...]


def _gcn_mid_call(parts, y, dinv, b, w2):
    full = lambda shape: pl.BlockSpec(shape, lambda t, i: tuple(0 for _ in shape))
    return pl.pallas_call(
        _k_gcn_mid,
        grid=(T, NPAD // NB),
        in_specs=[
            pl.BlockSpec((NC, 1, NB, H_N), lambda t, i: (0, t, i, 0)),
            pl.BlockSpec((1, NB, H_N), lambda t, i: (t, i, 0)),
            pl.BlockSpec((NB, 1), lambda t, i: (i, 0)),
            full((1, H_N)), full((H_N, H_N)),
        ],
        out_specs=pl.BlockSpec((1, NB, H_N), lambda t, i: (t, i, 0)),
        out_shape=jax.ShapeDtypeStruct((T, NPAD, H_N), F32),
    )(parts, y, dinv, b, w2)


def _k_gcn_out(parts_ref, y_ref, dinv_ref, b_ref, wih_ref, bsum_ref, out_ref):
    ssum = parts_ref[0, 0] + parts_ref[1, 0] + y_ref[0]
    x = jnp.maximum(ssum * dinv_ref[...] + b_ref[...], 0.0)
    out_ref[0] = _mm(x, wih_ref[...]) + bsum_ref[...]


def _gcn_out_call(parts, y, dinv, b, wih, bsum):
    full = lambda shape: pl.BlockSpec(shape, lambda t, i: tuple(0 for _ in shape))
    return pl.pallas_call(
        _k_gcn_out,
        grid=(T, NPAD // NB),
        in_specs=[
            pl.BlockSpec((NC, 1, NB, H_N), lambda t, i: (0, t, i, 0)),
            pl.BlockSpec((1, NB, H_N), lambda t, i: (t, i, 0)),
            pl.BlockSpec((NB, 1), lambda t, i: (i, 0)),
            full((1, H_N)), full((H_N, 4 * H_N)), full((1, 4 * H_N)),
        ],
        out_specs=pl.BlockSpec((1, NB, 4 * H_N), lambda t, i: (t, i, 0)),
        out_shape=jax.ShapeDtypeStruct((TP, NPAD, 4 * H_N), F32),
    )(parts, y, dinv, b, wih, bsum)


def _k_edge_front(x_ref, w1_ref, b1_ref, w2_ref, b2_ref, f1_ref, fb1_ref,
                  f2_ref, fb2_ref, wih_ref, bsum_ref, out_ref):
    x = x_ref[0]
    h = jnp.maximum(_mm(x, w1_ref[...]) + b1_ref[...], 0.0)
    h = jnp.maximum(_mm(h, w2_ref[...]) + b2_ref[...], 0.0)
    h = jnp.maximum(_mm(h, f1_ref[...]) + fb1_ref[...], 0.0)
    h = jnp.maximum(_mm(h, f2_ref[...]) + fb2_ref[...], 0.0)
    out_ref[0] = _mm(h, wih_ref[...]) + bsum_ref[...]


def _edge_front_call(ea, w1, b1, w2, b2, f1, fb1, f2, fb2, wih, bsum):
    full = lambda shape: pl.BlockSpec(shape, lambda t, i: tuple(0 for _ in shape))
    return pl.pallas_call(
        _k_edge_front,
        grid=(T, E // EB),
        in_specs=[
            pl.BlockSpec((1, EB, D_EDGE), lambda t, i: (t, i, 0)),
            full((D_EDGE, H_E)), full((1, H_E)),
            full((H_E, H_E)), full((1, H_E)),
            full((H_E, H_E)), full((1, H_E)),
            full((H_E, H_E)), full((1, H_E)),
            full((H_E, 4 * H_E)), full((1, 4 * H_E)),
        ],
        out_specs=pl.BlockSpec((1, EB, 4 * H_E), lambda t, i: (t, i, 0)),
        out_shape=jax.ShapeDtypeStruct((TP, E, 4 * H_E), F32),
    )(ea, w1, b1, w2, b2, f1, fb1, f2, fb2, wih, bsum)


def _make_lstm_kernel(h_dim, n_steps):
    def body(ih_ref, whh_ref, out_ref, h_ref, c_ref):
        @pl.when(pl.program_id(0) == 0)
        def _init():
            h_ref[...] = jnp.zeros((TP, h_dim), F32)
            c_ref[...] = jnp.zeros((TP, h_dim), F32)

        whh = whh_ref[...]

        def step(i, hc):
            h, c = hc
            g = ih_ref[:, i, :] + jax.lax.dot_general(
                h.astype(jnp.bfloat16), whh, (((1,), (0,)), ((), ())),
                preferred_element_type=F32)
            tt = jnp.tanh(g)
            ii = tt[:, :h_dim] + 1.0
            ff = tt[:, h_dim:2 * h_dim] + 1.0
            gg = tt[:, 2 * h_dim:3 * h_dim]
            oo = tt[:, 3 * h_dim:] + 1.0
            c = 0.5 * ff * c + 0.5 * ii * gg
            h = (0.5 * oo) * jnp.tanh(c)
            out_ref[:, pl.ds(i, 1), :] = h[:, None, :]
            return (h, c)

        hN, cN = lax.fori_loop(0, n_steps, step, (h_ref[...], c_ref[...]),
                               unroll=4)
        h_ref[...] = hN
        c_ref[...] = cN

    return body


def _lstm_call(ih, whh, h_dim, n_steps, total):
    g4 = 4 * h_dim
    return pl.pallas_call(
        _make_lstm_kernel(h_dim, n_steps),
        grid=(total // n_steps,),
        in_specs=[
            pl.BlockSpec((TP, n_steps, g4), lambda i: (0, i, 0)),
            pl.BlockSpec((h_dim, g4), lambda i: (0, 0)),
        ],
        out_specs=pl.BlockSpec((TP, n_steps, h_dim), lambda i: (0, i, 0)),
        out_shape=jax.ShapeDtypeStruct((TP, total, h_dim), F32),
        scratch_shapes=[
            pltpu.VMEM((TP, h_dim), F32),
            pltpu.VMEM((TP, h_dim), F32),
        ],
        compiler_params=pltpu.CompilerParams(
            dimension_semantics=("arbitrary",)),
    )(ih, whh)


def _k_class(gs_ref, gd_ref, le_ref, wa_ref, wb_ref, wc_ref, b1_ref,
             w2_ref, b2_ref, out_ref):
    acc = _mm(gs_ref[0], wa_ref[...])
    acc = acc + _mm(gd_ref[0], wb_ref[...])
    acc = acc + _mm(le_ref[0], wc_ref[...])
    h = jnp.maximum(acc + b1_ref[...], 0.0)
    out_ref[0] = _mm(h, w2_ref[...]) + b2_ref[...]


def _class_call(gs, gd, lep, wa, wb, wc, b1, w2, b2):
    full = lambda shape: pl.BlockSpec(shape, lambda t, i: tuple(0 for _ in shape))
    return pl.pallas_call(
        _k_class,
        grid=(T, EPAD // EBP),
        in_specs=[
            pl.BlockSpec((1, EBP, H_N), lambda t, i: (t, i, 0)),
            pl.BlockSpec((1, EBP, H_N), lambda t, i: (t, i, 0)),
            pl.BlockSpec((1, EBP, H_E), lambda t, i: (t, i, 0)),
            full((H_N, MLP_H)), full((H_N, MLP_H)), full((H_E, MLP_H)),
            full((1, MLP_H)), full((MLP_H, N_CLS)), full((1, N_CLS)),
        ],
        out_specs=pl.BlockSpec((1, EBP, N_CLS), lambda t, i: (t, i, 0)),
        out_shape=jax.ShapeDtypeStruct((T, EPAD, N_CLS), F32),
    )(gs, gd, lep, wa, wb, wc, b1, w2, b2)


def kernel(x_seq, edge_attr_seq, edge_index, seq_len, W_node1, b_node1,
           W_node2, b_node2, W_edge1, b_edge1, W_edge2, b_edge2, gcn1_W,
           gcn1_b, gcn2_W, gcn2_b, efc1_W, efc1_b, efc2_W, efc2_b, Wih_n,
           Whh_n, bih_n, bhh_n, Wih_e, Whh_e, bih_e, bhh_e, Wc1, bc1, Wc2,
           bc2):
    src = edge_index[0].astype(jnp.int32)
    dst = edge_index[1].astype(jnp.int32)
    src_p = jnp.pad(src, (0, EPAD - E)).reshape(NW, KSUB, CHB)
    dst_p = jnp.pad(dst, (0, EPAD - E),
                    constant_values=N).reshape(NW, KSUB, CHB)

    ones128 = jnp.ones((CHB, H_N), F32)
    zeros128 = jnp.zeros((STRIPE, H_N), F32)

    deg_parts = _get_sc_degree()(dst_p, ones128, zeros128)
    dinv = _dinv_call(deg_parts)

    xp = jnp.pad(x_seq, ((0, 0), (0, NPAD - N), (0, 0)))
    y1 = _node_front_call(xp, W_node1, b_node1.reshape(1, -1), W_node2,
                          b_node2.reshape(1, -1), gcn1_W, dinv)
    p1 = _get_sc_gcn()(y1[0], y1[1], y1[2], y1[3], src_p, dst_p, zeros128)
    y2 = _gcn_mid_call(p1, y1, dinv, gcn1_b.reshape(1, -1), gcn2_W)
    p2 = _get_sc_gcn()(y2[0], y2[1], y2[2], y2[3], src_p, dst_p, zeros128)
    bsum_n = _gate_scale((bih_n + bhh_n), H_N).reshape(1, -1)
    wih_n = _gate_scale(Wih_n, H_N)
    whh_n = _gate_scale(Whh_n, H_N).astype(jnp.bfloat16)
    ih_n = _gcn_out_call(p2, y2, dinv, gcn2_b.reshape(1, -1), wih_n, bsum_n)
    ln = _lstm_call(ih_n, whh_n, H_N, NBL, NPAD)

    bsum_e = _gate_scale((bih_e + bhh_e), H_E).reshape(1, -1)
    wih_e = _gate_scale(Wih_e, H_E)
    whh_e = _gate_scale(Whh_e, H_E).astype(jnp.bfloat16)
    ih_e = _edge_front_call(edge_attr_seq, W_edge1, b_edge1.reshape(1, -1),
                            W_edge2, b_edge2.reshape(1, -1), efc1_W,
                            efc1_b.reshape(1, -1), efc2_W,
                            efc2_b.reshape(1, -1), wih_e, bsum_e)
    le = _lstm_call(ih_e, whh_e, H_E, EBL, E)

    gs, gd = _get_sc_class_gather()(ln[0], ln[1], ln[2], ln[3], src_p, dst_p)
    le_p = jnp.pad(le[:T], ((0, 0), (0, EPAD - E), (0, 0)))
    preds_p = _class_call(gs, gd, le_p, Wc1[:H_N], Wc1[H_N:2 * H_N],
                          Wc1[2 * H_N:], bc1.reshape(1, -1), Wc2,
                          bc2.reshape(1, -1))
    return preds_p[:, :E, :]
